# Initial kernel scaffold; baseline (speedup 1.0000x reference)
import jax
import jax.numpy as jnp
from jax import lax
from jax.experimental import pallas as pl
from jax.experimental.pallas import tpu as pltpu
from jax.experimental.pallas import tpu_sc as plsc

N = 10000
E = 320000
D_IN = 128
D_HID = 128
D_OUT = 64

NC = 2
NS = 16
NW = NC * NS
Np = 10240
K = 128
NCHUNK = 80
EPT = NCHUNK * K
EP = NW * EPT
ROWS_PT = Np // NS
BM = 512

_MESH = dict(core_axis_name="c", subcore_axis_name="s")


def _deg_body(dst_hbm, out_hbm, dst_v, hist_v):
    cid = lax.axis_index("c")
    sid = lax.axis_index("s")
    wid = cid * NS + sid
    pltpu.sync_copy(dst_hbm.at[wid], dst_v)
    zero16 = jnp.zeros((16,), jnp.float32)
    ones16 = jnp.full((16,), 1.0, jnp.float32)

    def z(i, c):
        hist_v[pl.ds(i * 16, 16)] = zero16
        return c

    lax.fori_loop(0, Np // 16, z, 0)

    def h(i, c):
        plsc.addupdate_scatter(hist_v, [dst_v[i]], ones16)
        return c

    lax.fori_loop(0, EPT // 16, h, 0)
    pltpu.sync_copy(hist_v, out_hbm.at[wid])


_deg_call = pl.kernel(
    _deg_body,
    out_type=jax.ShapeDtypeStruct((NW, Np), jnp.float32),
    mesh=plsc.VectorSubcoreMesh(**_MESH),
    scratch_types=[
        pltpu.VMEM((EPT // 16, 16), jnp.int32),
        pltpu.VMEM((Np,), jnp.float32),
    ],
)


def _make_agg(D):
    def body(y_hbm, src_hbm, dst_hbm, out_hbm, src_v, dst_v, rows_v, acc_sh, sem):
        cid = lax.axis_index("c")
        sid = lax.axis_index("s")
        wid = cid * NS + sid
        pltpu.sync_copy(src_hbm.at[wid], src_v)
        pltpu.sync_copy(dst_hbm.at[wid], dst_v)
        zero16 = jnp.zeros((16,), jnp.float32)

        def z(i, c):
            for j in range(D // 16):
                rows_v[i, pl.ds(j * 16, 16)] = zero16
            return c

        lax.fori_loop(0, K, z, 0)
        for j in range(ROWS_PT // K):
            pltpu.sync_copy(rows_v, acc_sh.at[pl.ds(sid * ROWS_PT + j * K, K)])
        plsc.subcore_barrier()

        def b(j, c):
            pltpu.async_copy(y_hbm.at[src_v.at[j]], rows_v, sem).wait()
            pltpu.sync_copy(rows_v, acc_sh.at[dst_v.at[j]], add=True)
            return c

        lax.fori_loop(0, NCHUNK, b, 0)
        plsc.subcore_barrier()
        for j in range(ROWS_PT // K):
            sl = pl.ds(sid * ROWS_PT + j * K, K)
            pltpu.sync_copy(acc_sh.at[sl], out_hbm.at[cid, sl])

    return pl.kernel(
        body,
        out_type=jax.ShapeDtypeStruct((NC, Np, D), jnp.float32),
        mesh=plsc.VectorSubcoreMesh(**_MESH),
        scratch_types=[
            pltpu.VMEM((NCHUNK, K), jnp.int32),
            pltpu.VMEM((NCHUNK, K), jnp.int32),
            pltpu.VMEM((K, D), jnp.float32),
            pltpu.VMEM_SHARED((Np, D), jnp.float32),
            pltpu.SemaphoreType.DMA,
        ],
    )


_agg_hid = _make_agg(D_HID)
_agg_out = _make_agg(D_OUT)


def _dis_body(degp_ref, dis_ref):
    deg = jnp.sum(degp_ref[...], axis=0).reshape(1, Np) + 1.0
    dis_ref[...] = lax.rsqrt(deg)


_dis_call = pl.pallas_call(
    _dis_body,
    out_shape=jax.ShapeDtypeStruct((1, Np), jnp.float32),
)


def _mm1_body(x_ref, w_ref, dis_ref, y_ref):
    i = pl.program_id(0)
    xw = jnp.dot(x_ref[...], w_ref[...], preferred_element_type=jnp.float32)
    d = dis_ref[0, pl.ds(i * BM, BM)]
    y_ref[...] = xw * d[:, None]


_mm1_call = pl.pallas_call(
    _mm1_body,
    grid=(Np // BM,),
    in_specs=[
        pl.BlockSpec((BM, D_IN), lambda i: (i, 0)),
        pl.BlockSpec((D_IN, D_HID), lambda i: (0, 0)),
        pl.BlockSpec((1, Np), lambda i: (0, 0)),
    ],
    out_specs=pl.BlockSpec((BM, D_HID), lambda i: (i, 0)),
    out_shape=jax.ShapeDtypeStruct((Np, D_HID), jnp.float32),
)


def _mid_body(p_ref, y1_ref, dis_ref, b1_ref, w2_ref, y2_ref):
    i = pl.program_id(0)
    d = dis_ref[0, pl.ds(i * BM, BM)][:, None]
    h = d * (p_ref[0] + p_ref[1] + y1_ref[...]) + b1_ref[...]
    h = jnp.where(h > 0, h, jnp.expm1(h))
    y2_ref[...] = jnp.dot(h, w2_ref[...], preferred_element_type=jnp.float32) * d


_mid_call = pl.pallas_call(
    _mid_body,
    grid=(Np // BM,),
    in_specs=[
        pl.BlockSpec((NC, BM, D_HID), lambda i: (0, i, 0)),
        pl.BlockSpec((BM, D_HID), lambda i: (i, 0)),
        pl.BlockSpec((1, Np), lambda i: (0, 0)),
        pl.BlockSpec((1, D_HID), lambda i: (0, 0)),
        pl.BlockSpec((D_HID, D_OUT), lambda i: (0, 0)),
    ],
    out_specs=pl.BlockSpec((BM, D_OUT), lambda i: (i, 0)),
    out_shape=jax.ShapeDtypeStruct((Np, D_OUT), jnp.float32),
)


def _fin_body(q_ref, y2_ref, dis_ref, b2_ref, o_ref):
    i = pl.program_id(0)
    d = dis_ref[0, pl.ds(i * BM, BM)][:, None]
    o_ref[...] = d * (q_ref[0] + q_ref[1] + y2_ref[...]) + b2_ref[...]


_fin_call = pl.pallas_call(
    _fin_body,
    grid=(Np // BM,),
    in_specs=[
        pl.BlockSpec((NC, BM, D_OUT), lambda i: (0, i, 0)),
        pl.BlockSpec((BM, D_OUT), lambda i: (i, 0)),
        pl.BlockSpec((1, Np), lambda i: (0, 0)),
        pl.BlockSpec((1, D_OUT), lambda i: (0, 0)),
    ],
    out_specs=pl.BlockSpec((BM, D_OUT), lambda i: (i, 0)),
    out_shape=jax.ShapeDtypeStruct((Np, D_OUT), jnp.float32),
)


def kernel(x, edge_index, W1, b1, W2, b2):
    xp = jnp.pad(x, ((0, Np - N), (0, 0)))
    pad = jnp.full((EP - E,), N, jnp.int32)
    srcp = jnp.concatenate([edge_index[0], pad]).reshape(NW, NCHUNK, K)
    dst_flat = jnp.concatenate([edge_index[1], pad])
    dstp = dst_flat.reshape(NW, NCHUNK, K)

    degp = _deg_call(dst_flat.reshape(NW, EPT // 16, 16))
    dis = _dis_call(degp)
    y1 = _mm1_call(xp, W1, dis)
    p = _agg_hid(y1, srcp, dstp)
    y2 = _mid_call(p, y1, dis, b1.reshape(1, D_HID), W2)
    q = _agg_out(y2, srcp, dstp)
    out = _fin_call(q, y2, dis, b2.reshape(1, D_OUT))
    return out[:N]


# speedup vs baseline: 8.4947x; 8.4947x over previous
import jax
import jax.numpy as jnp
from jax import lax
from jax.experimental import pallas as pl
from jax.experimental.pallas import tpu as pltpu
from jax.experimental.pallas import tpu_sc as plsc

N = 10000
E = 320000
D_IN = 128
D_HID = 128
D_OUT = 64

NC = 2
NS = 16
NW = NC * NS
Np = 10240
K = 128
NCHUNK = 80
EPT = NCHUNK * K
EP = NW * EPT
ROWS_PT = Np // NS
BM = 512

_MESH = dict(core_axis_name="c", subcore_axis_name="s")


def _deg_body(dst_hbm, out_hbm, dst_v, hist_v):
    cid = lax.axis_index("c")
    sid = lax.axis_index("s")
    wid = cid * NS + sid
    pltpu.sync_copy(dst_hbm.at[wid], dst_v)
    zero16 = jnp.zeros((16,), jnp.float32)
    ones16 = jnp.full((16,), 1.0, jnp.float32)

    def z(i, c):
        hist_v[pl.ds(i * 16, 16)] = zero16
        return c

    lax.fori_loop(0, Np // 16, z, 0)

    def h(i, c):
        plsc.addupdate_scatter(hist_v, [dst_v[i]], ones16)
        return c

    lax.fori_loop(0, EPT // 16, h, 0)
    pltpu.sync_copy(hist_v, out_hbm.at[wid])


_deg_call = pl.kernel(
    _deg_body,
    out_type=jax.ShapeDtypeStruct((NW, Np), jnp.float32),
    mesh=plsc.VectorSubcoreMesh(**_MESH),
    compiler_params=pltpu.CompilerParams(needs_layout_passes=False),
    scratch_types=[
        pltpu.VMEM((EPT // 16, 16), jnp.int32),
        pltpu.VMEM((Np,), jnp.float32),
    ],
)


def _make_agg(D):
    def body(y_hbm, src_hbm, dst_hbm, out_hbm, src_v, dst_v, rows_v, acc_sh, sem):
        cid = lax.axis_index("c")
        sid = lax.axis_index("s")
        wid = cid * NS + sid
        pltpu.sync_copy(src_hbm.at[wid], src_v)
        pltpu.sync_copy(dst_hbm.at[wid], dst_v)
        zero16 = jnp.zeros((16,), jnp.float32)

        def z(i, c):
            for j in range(D // 16):
                rows_v[i, pl.ds(j * 16, 16)] = zero16
            return c

        lax.fori_loop(0, K, z, 0)
        for j in range(ROWS_PT // K):
            pltpu.sync_copy(rows_v, acc_sh.at[pl.ds(sid * ROWS_PT + j * K, K)])
        plsc.subcore_barrier()

        def b(j, c):
            pltpu.async_copy(y_hbm.at[src_v.at[j]], rows_v, sem).wait()
            pltpu.sync_copy(rows_v, acc_sh.at[dst_v.at[j]], add=True)
            return c

        lax.fori_loop(0, NCHUNK, b, 0)
        plsc.subcore_barrier()
        for j in range(ROWS_PT // K):
            sl = pl.ds(sid * ROWS_PT + j * K, K)
            pltpu.sync_copy(acc_sh.at[sl], out_hbm.at[cid, sl])

    return pl.kernel(
        body,
        out_type=jax.ShapeDtypeStruct((NC, Np, D), jnp.float32),
        mesh=plsc.VectorSubcoreMesh(**_MESH),
        scratch_types=[
            pltpu.VMEM((NCHUNK, K), jnp.int32),
            pltpu.VMEM((NCHUNK, K), jnp.int32),
            pltpu.VMEM((K, D), jnp.float32),
            pltpu.VMEM_SHARED((Np, D), jnp.float32),
            pltpu.SemaphoreType.DMA,
        ],
    )


_agg_hid = _make_agg(D_HID)


def _dis_body(degp_ref, dis_ref):
    deg = jnp.sum(degp_ref[...], axis=0).reshape(1, Np) + 1.0
    dis_ref[...] = lax.rsqrt(deg)


_dis_call = pl.pallas_call(
    _dis_body,
    out_shape=jax.ShapeDtypeStruct((1, Np), jnp.float32),
)


def _mm1_body(x_ref, w_ref, dis_ref, y_ref):
    i = pl.program_id(0)
    xw = jnp.dot(x_ref[...], w_ref[...], preferred_element_type=jnp.float32)
    d = dis_ref[0, pl.ds(i * BM, BM)]
    y_ref[...] = xw * d[:, None]


_mm1_call = pl.pallas_call(
    _mm1_body,
    grid=(Np // BM,),
    in_specs=[
        pl.BlockSpec((BM, D_IN), lambda i: (i, 0)),
        pl.BlockSpec((D_IN, D_HID), lambda i: (0, 0)),
        pl.BlockSpec((1, Np), lambda i: (0, 0)),
    ],
    out_specs=pl.BlockSpec((BM, D_HID), lambda i: (i, 0)),
    out_shape=jax.ShapeDtypeStruct((Np, D_HID), jnp.float32),
)


def _mid_body(p_ref, y1_ref, dis_ref, b1_ref, z_ref):
    i = pl.program_id(0)
    d = dis_ref[0, pl.ds(i * BM, BM)][:, None]
    h = d * (p_ref[0] + p_ref[1] + y1_ref[...]) + b1_ref[...]
    h = jnp.where(h > 0, h, jnp.exp(jnp.minimum(h, 0.0)) - 1.0)
    z_ref[...] = h * d


_mid_call = pl.pallas_call(
    _mid_body,
    grid=(Np // BM,),
    in_specs=[
        pl.BlockSpec((NC, BM, D_HID), lambda i: (0, i, 0)),
        pl.BlockSpec((BM, D_HID), lambda i: (i, 0)),
        pl.BlockSpec((1, Np), lambda i: (0, 0)),
        pl.BlockSpec((1, D_HID), lambda i: (0, 0)),
    ],
    out_specs=pl.BlockSpec((BM, D_HID), lambda i: (i, 0)),
    out_shape=jax.ShapeDtypeStruct((Np, D_HID), jnp.float32),
)


def _fin_body(q_ref, z_ref, dis_ref, w2_ref, b2_ref, o_ref):
    i = pl.program_id(0)
    d = dis_ref[0, pl.ds(i * BM, BM)][:, None]
    agg = d * (q_ref[0] + q_ref[1] + z_ref[...])
    o_ref[...] = (
        jnp.dot(agg, w2_ref[...], preferred_element_type=jnp.float32) + b2_ref[...]
    )


_fin_call = pl.pallas_call(
    _fin_body,
    grid=(Np // BM,),
    in_specs=[
        pl.BlockSpec((NC, BM, D_HID), lambda i: (0, i, 0)),
        pl.BlockSpec((BM, D_HID), lambda i: (i, 0)),
        pl.BlockSpec((1, Np), lambda i: (0, 0)),
        pl.BlockSpec((D_HID, D_OUT), lambda i: (0, 0)),
        pl.BlockSpec((1, D_OUT), lambda i: (0, 0)),
    ],
    out_specs=pl.BlockSpec((BM, D_OUT), lambda i: (i, 0)),
    out_shape=jax.ShapeDtypeStruct((Np, D_OUT), jnp.float32),
)


def kernel(x, edge_index, W1, b1, W2, b2):
    xp = jnp.pad(x, ((0, Np - N), (0, 0)))
    pad = jnp.full((EP - E,), N, jnp.int32)
    srcp = jnp.concatenate([edge_index[0], pad]).reshape(NW, NCHUNK, K)
    dst_flat = jnp.concatenate([edge_index[1], pad])
    dstp = dst_flat.reshape(NW, NCHUNK, K)

    degp = _deg_call(dst_flat.reshape(NW, EPT // 16, 16))
    dis = _dis_call(degp)
    y1 = _mm1_call(xp, W1, dis)
    p = _agg_hid(y1, srcp, dstp)
    z = _mid_call(p, y1, dis, b1.reshape(1, D_HID))
    q = _agg_hid(z, srcp, dstp)
    out = _fin_call(q, z, dis, W2, b2.reshape(1, D_OUT))
    return out[:N]
